# baseline (speedup 1.0000x reference)
import functools
import jax
import jax.numpy as jnp
from jax.experimental import pallas as pl
from jax.experimental.pallas import tpu as pltpu
from jax.experimental.pallas import tpu_sc as plsc

_VOCAB = 10
_HIDDEN = 1024
_W = 32


def kernel(indices, word_embeddings):
    n_rows, n_cols = indices.shape
    total = n_rows * n_cols
    idx = indices.astype(jnp.int32).reshape(1, total)
    mesh = plsc.VectorSubcoreMesh(core_axis_name="c", subcore_axis_name="s")

    @functools.partial(
        pl.kernel,
        out_type=jax.ShapeDtypeStruct((total, _HIDDEN), jnp.float32),
        mesh=mesh,
    )
    def lookup(table_hbm, idx_hbm, out_hbm):
        sub = 128 // _W

        def body(i_vmem, o_vmem):
            j = pl.program_id(1)
            pltpu.sync_copy(
                table_hbm.at[i_vmem.at[0, pl.ds(j * _W, _W)]], o_vmem
            )

        pltpu.emit_pipeline(
            body,
            grid=(total // 128, sub),
            in_specs=[pl.BlockSpec((1, 128), lambda i, j: (0, i))],
            out_specs=[
                pl.BlockSpec((_W, _HIDDEN), lambda i, j: (i * sub + j, 0))
            ],
            core_axis_name=("c", "s"),
            dimension_semantics=(pltpu.PARALLEL, pltpu.ARBITRARY),
        )(idx_hbm, out_hbm)

    out = lookup(word_embeddings, idx)
    return out.reshape(n_rows, n_cols, _HIDDEN)


# speedup vs baseline: 1.0021x; 1.0021x over previous
import functools
import jax
import jax.numpy as jnp
from jax import lax
from jax.experimental import pallas as pl
from jax.experimental.pallas import tpu as pltpu
from jax.experimental.pallas import tpu_sc as plsc

_VOCAB = 10
_HIDDEN = 1024
_NW = 32
_C = 40


def kernel(indices, word_embeddings):
    n_rows, n_cols = indices.shape
    total = n_rows * n_cols
    bpw = total // _NW
    nch = bpw // _C
    idx = indices.astype(jnp.int32).reshape(total)
    mesh = plsc.VectorSubcoreMesh(core_axis_name="c", subcore_axis_name="s")

    @functools.partial(
        pl.kernel,
        out_type=jax.ShapeDtypeStruct((total, _HIDDEN), jnp.float32),
        mesh=mesh,
        scratch_types=[
            pltpu.VMEM((bpw,), jnp.int32),
            pltpu.VMEM((_C, _HIDDEN), jnp.float32),
            pltpu.VMEM((_C, _HIDDEN), jnp.float32),
            pltpu.SemaphoreType.DMA,
            pltpu.SemaphoreType.DMA,
            pltpu.SemaphoreType.DMA,
            pltpu.SemaphoreType.DMA,
        ],
    )
    def lookup(table_hbm, idx_hbm, out_hbm, idx_v, buf0, buf1, sg0, sg1, sw0, sw1):
        wid = lax.axis_index("s") * 2 + lax.axis_index("c")
        base = wid * bpw
        pltpu.sync_copy(idx_hbm.at[pl.ds(base, bpw)], idx_v)

        def fire_gather(c, buf, sem):
            pltpu.async_copy(table_hbm.at[idx_v.at[pl.ds(c * _C, _C)]], buf, sem)

        def wait_gather(buf, sem):
            pltpu.make_async_copy(
                table_hbm.at[idx_v.at[pl.ds(0, _C)]], buf, sem
            ).wait()

        def fire_write(c, buf, sem):
            pltpu.async_copy(buf, out_hbm.at[pl.ds(base + c * _C, _C)], sem)

        def wait_write(buf, sem):
            pltpu.make_async_copy(buf, out_hbm.at[pl.ds(base, _C)], sem).wait()

        fire_gather(0, buf0, sg0)

        @pl.loop(0, nch, step=2)
        def _(c):
            @pl.when(c > 0)
            def _():
                wait_write(buf1, sw1)

            fire_gather(c + 1, buf1, sg1)
            wait_gather(buf0, sg0)
            fire_write(c, buf0, sw0)

            @pl.when(c + 2 < nch)
            def _():
                wait_write(buf0, sw0)
                fire_gather(c + 2, buf0, sg0)

            wait_gather(buf1, sg1)
            fire_write(c + 1, buf1, sw1)

        wait_write(buf0, sw0)
        wait_write(buf1, sw1)

    out = lookup(word_embeddings, idx)
    return out.reshape(n_rows, n_cols, _HIDDEN)


# speedup vs baseline: 3.9438x; 3.9356x over previous
import jax
import jax.numpy as jnp
from jax.experimental import pallas as pl

_VOCAB = 10
_HIDDEN = 1024
_B = 128


def _lookup_block(idx_ref, table_ref, out_ref):
    idx = idx_ref[...]
    cond = idx[:, :, None]
    r = jnp.broadcast_to(
        table_ref[0, :][None, None, :], (_B, idx.shape[1], _HIDDEN)
    )
    for v in range(1, _VOCAB):
        r = jnp.where(cond == v, table_ref[v, :][None, None, :], r)
    out_ref[...] = r


def kernel(indices, word_embeddings):
    n_rows, n_cols = indices.shape
    idx = indices.astype(jnp.int32)

    return pl.pallas_call(
        _lookup_block,
        grid=(n_rows // _B,),
        in_specs=[
            pl.BlockSpec((_B, n_cols), lambda i: (i, 0)),
            pl.BlockSpec((_VOCAB, _HIDDEN), lambda i: (0, 0)),
        ],
        out_specs=pl.BlockSpec((_B, n_cols, _HIDDEN), lambda i: (i, 0, 0)),
        out_shape=jax.ShapeDtypeStruct((n_rows, n_cols, _HIDDEN), jnp.float32),
    )(idx, word_embeddings)


# speedup vs baseline: 13.1262x; 3.3283x over previous
import jax
import jax.numpy as jnp
from jax.experimental import pallas as pl

_VOCAB = 10
_HIDDEN = 1024
_B = 128


def _lookup_block(idx_ref, table_ref, out_ref):
    idx = idx_ref[...]
    cond = idx[:, :, None]
    r = jnp.broadcast_to(
        table_ref[0, :][None, None, :], (idx.shape[0], _B, _HIDDEN)
    )
    for v in range(1, _VOCAB):
        r = jnp.where(cond == v, table_ref[v, :][None, None, :], r)
    out_ref[...] = r


def kernel(indices, word_embeddings):
    n_rows, n_cols = indices.shape
    idx_t = jnp.swapaxes(indices.astype(jnp.int32), 0, 1)

    out_t = pl.pallas_call(
        _lookup_block,
        grid=(n_rows // _B,),
        in_specs=[
            pl.BlockSpec((n_cols, _B), lambda i: (0, i)),
            pl.BlockSpec((_VOCAB, _HIDDEN), lambda i: (0, 0)),
        ],
        out_specs=pl.BlockSpec((n_cols, _B, _HIDDEN), lambda i: (0, i, 0)),
        out_shape=jax.ShapeDtypeStruct((n_cols, n_rows, _HIDDEN), jnp.float32),
    )(idx_t, word_embeddings)
    return jnp.swapaxes(out_t, 0, 1)
